# baseline (speedup 1.0000x reference)
import functools

import jax
import jax.numpy as jnp
from jax import lax
from jax.experimental import pallas as pl
from jax.experimental.pallas import tpu as pltpu
from jax.experimental.pallas import tpu_sc as plsc

VOCAB = 1000
VPAD = 1024
TOKENS = 1024 * 50

NC = 2
NS = 16
NW = NC * NS
L = 16
TPW = TOKENS // NW
NB = 16
NBLK = TPW // NB


def _lse_body(t_ref, o_ref):
    t = t_ref[...]
    m = jnp.max(t, axis=1, keepdims=True)
    s = jnp.sum(jnp.exp(t - m), axis=1)
    o_ref[...] = m[:, 0] + jnp.log(s)


def _lse_table(table):
    return pl.pallas_call(
        _lse_body,
        out_shape=jax.ShapeDtypeStruct((VOCAB,), jnp.float32),
    )(table)


def _sc_body(table3, tflat, idxs, tgts, lse, out, partials,
             idx_v, tgt_v, lse_v, pick_v, picked_v,
             r3_0, r3_1, rk_0, rk_1, acc_v, psem, g0, g1, o0, o1):
    wid = lax.axis_index("s") * NC + lax.axis_index("c")
    base = pl.multiple_of(wid * TPW, 8)
    pltpu.sync_copy(idxs.at[pl.ds(base, TPW)], idx_v)
    pltpu.sync_copy(tgts.at[pl.ds(base, TPW)], tgt_v)
    pltpu.sync_copy(lse, lse_v)

    rows3 = (r3_0, r3_1)
    rowsk = (rk_0, rk_1)
    gs = (g0, g1)
    os_ = (o0, o1)

    def g_desc(b, buf):
        src = table3.at[idx_v.at[pl.ds(pl.multiple_of(b * NB, 8), NB)]]
        return src, rows3[buf], gs[buf]

    def s_desc(b, buf):
        dst = out.at[pl.ds(pl.multiple_of(base + b * NB, 8), NB)]
        return rowsk[buf], dst, os_[buf]

    def pgroup(g, c):
        off = pl.multiple_of(g * L, 8)
        i16 = idx_v[pl.ds(off, L)]
        t16 = tgt_v[pl.ds(off, L)]
        pick_v[pl.ds(off, L)] = i16 * VOCAB + t16
        return c

    lax.fori_loop(0, TPW // L, pgroup, 0)

    chunks = [(c * 128, 128) for c in range(TPW // 128)]
    if TPW % 128:
        chunks.append((TPW - TPW % 128, TPW % 128))
    for off, n in chunks:
        off = pl.multiple_of(off, 8)
        pltpu.async_copy(tflat.at[pick_v.at[pl.ds(off, n)]],
                         picked_v.at[pl.ds(off, n)], psem)

    pltpu.async_copy(*g_desc(0, 0))
    pltpu.async_copy(*g_desc(1, 1))

    for off, n in chunks:
        off = pl.multiple_of(off, 8)
        pltpu.make_async_copy(tflat.at[pick_v.at[pl.ds(off, n)]],
                              picked_v.at[pl.ds(off, n)], psem).wait()

    def lgroup(g, acc):
        off = pl.multiple_of(g * L, 8)
        i16 = idx_v[pl.ds(off, L)]
        l16 = plsc.load_gather(lse_v, [i16])
        return acc + (l16 - picked_v[pl.ds(off, L)])

    acc = lax.fori_loop(0, TPW // L, lgroup, jnp.zeros((L,), jnp.float32))
    acc_v[...] = acc
    pltpu.sync_copy(acc_v, partials.at[pl.ds(wid * L, L)])

    def rechunk(buf):
        r3 = rows3[buf]
        rk = rowsk[buf]

        for t in range(NB):
            for ch in range(62):
                rk[t, pl.ds(ch * L, L)] = r3[t, ch >> 3, pl.ds((ch & 7) * L, L)]
            rk[t, pl.ds(984, L)] = r3[t, 7, pl.ds(88, L)]

    def step(b, buf, wait_s2, start_next):
        pltpu.make_async_copy(*g_desc(b, buf)).wait()
        if start_next:
            pltpu.async_copy(*g_desc(b + 1, 1 - buf))
        if wait_s2:
            pltpu.make_async_copy(*s_desc(b - 2, buf)).wait()
        rechunk(buf)
        pltpu.async_copy(*s_desc(b, buf))

    step(0, 0, False, False)
    step(1, 1, False, True)
    step(2, 0, True, True)

    def pair(j, c):
        step(2 * j + 3, 1, True, True)
        step(2 * j + 4, 0, True, True)
        return c

    lax.fori_loop(0, (NBLK - 4) // 2, pair, 0)

    step(NBLK - 1, 1, True, False)
    pltpu.make_async_copy(*s_desc(NBLK - 2, 0)).wait()
    pltpu.make_async_copy(*s_desc(NBLK - 1, 1)).wait()


def _sc_main(table3, tflat, flat_idx, flat_tgt, lse):
    mesh = plsc.VectorSubcoreMesh(core_axis_name="c", subcore_axis_name="s")
    f = functools.partial(
        pl.kernel,
        mesh=mesh,
        compiler_params=pltpu.CompilerParams(needs_layout_passes=False),
        out_type=[
            jax.ShapeDtypeStruct((TOKENS, VOCAB), jnp.float32),
            jax.ShapeDtypeStruct((NW * L,), jnp.float32),
        ],
        scratch_types=[
            pltpu.VMEM((TPW,), jnp.int32),
            pltpu.VMEM((TPW,), jnp.int32),
            pltpu.VMEM((VOCAB,), jnp.float32),
            pltpu.VMEM((TPW,), jnp.int32),
            pltpu.VMEM((TPW,), jnp.float32),
            pltpu.VMEM((NB, 8, 128), jnp.float32),
            pltpu.VMEM((NB, 8, 128), jnp.float32),
            pltpu.VMEM((NB, VOCAB), jnp.float32),
            pltpu.VMEM((NB, VOCAB), jnp.float32),
            pltpu.VMEM((L,), jnp.float32),
            pltpu.SemaphoreType.DMA,
            pltpu.SemaphoreType.DMA,
            pltpu.SemaphoreType.DMA,
            pltpu.SemaphoreType.DMA,
            pltpu.SemaphoreType.DMA,
        ],
    )(_sc_body)
    return f(table3, tflat, flat_idx, flat_tgt, lse)


def kernel(idx, targets, token_embedding_table):
    flat_idx = idx.reshape(-1).astype(jnp.int32)
    flat_tgt = targets.reshape(-1).astype(jnp.int32)
    table3 = jnp.pad(
        token_embedding_table, ((0, 0), (0, VPAD - VOCAB))
    ).reshape(VOCAB, 8, 128)
    tflat = token_embedding_table.reshape(-1)
    lse = _lse_table(token_embedding_table)
    flat_logits, partials = _sc_main(table3, tflat, flat_idx, flat_tgt, lse)
    loss = jnp.sum(partials) / TOKENS
    return (flat_logits, loss)


# speedup vs baseline: 1.0221x; 1.0221x over previous
import functools

import jax
import jax.numpy as jnp
from jax import lax
from jax.experimental import pallas as pl
from jax.experimental.pallas import tpu as pltpu
from jax.experimental.pallas import tpu_sc as plsc

VOCAB = 1000
VPAD = 1024
TOKENS = 1024 * 50

NC = 2
NS = 16
NW = NC * NS
L = 16
TPW = TOKENS // NW
NB = 16
NBLK = TPW // NB


def _lse_body(t_ref, o_ref):
    t = t_ref[...]
    m = jnp.max(t, axis=1, keepdims=True)
    s = jnp.sum(jnp.exp(t - m), axis=1)
    o_ref[...] = m[:, 0] + jnp.log(s)


def _lse_table(table):
    return pl.pallas_call(
        _lse_body,
        out_shape=jax.ShapeDtypeStruct((VOCAB,), jnp.float32),
    )(table)


def _sc_body(table3, idxs, tgts, lse, out, partials,
             idx_v, tgt_v, lse_v,
             r3_0, r3_1, rk_0, rk_1, acc_v, g0, g1, o0, o1):
    wid = lax.axis_index("s") * NC + lax.axis_index("c")
    base = pl.multiple_of(wid * TPW, 8)
    pltpu.sync_copy(idxs.at[pl.ds(base, TPW)], idx_v)
    pltpu.sync_copy(tgts.at[pl.ds(base, TPW)], tgt_v)
    pltpu.sync_copy(lse, lse_v)

    rows3 = (r3_0, r3_1)
    rowsk = (rk_0, rk_1)
    gs = (g0, g1)
    os_ = (o0, o1)
    tok16 = lax.iota(jnp.int32, L)

    def g_desc(b, buf):
        src = table3.at[idx_v.at[pl.ds(pl.multiple_of(b * NB, 8), NB)]]
        return src, rows3[buf], gs[buf]

    def s_desc(b, buf):
        dst = out.at[pl.ds(pl.multiple_of(base + b * NB, 8), NB)]
        return rowsk[buf], dst, os_[buf]

    pltpu.async_copy(*g_desc(0, 0))
    pltpu.async_copy(*g_desc(1, 1))

    def rechunk(buf):
        r3 = rows3[buf]
        rk = rowsk[buf]

        for t in range(NB):
            for ch in range(62):
                rk[t, pl.ds(ch * L, L)] = r3[t, ch >> 3, pl.ds((ch & 7) * L, L)]
            rk[t, pl.ds(984, L)] = r3[t, 7, pl.ds(88, L)]

    def step(b, buf, wait_s2, start_next, acc):
        pltpu.make_async_copy(*g_desc(b, buf)).wait()
        if start_next:
            pltpu.async_copy(*g_desc(b + 1, 1 - buf))
        if wait_s2:
            pltpu.make_async_copy(*s_desc(b - 2, buf)).wait()
        off = pl.multiple_of(b * L, 8)
        i16 = idx_v[pl.ds(off, L)]
        t16 = tgt_v[pl.ds(off, L)]
        l16 = plsc.load_gather(lse_v, [i16])
        p16 = plsc.load_gather(
            rows3[buf],
            [tok16, lax.shift_right_logical(t16, 7), lax.bitwise_and(t16, 127)])
        acc = acc + (l16 - p16)
        rechunk(buf)
        pltpu.async_copy(*s_desc(b, buf))
        return acc

    acc = jnp.zeros((L,), jnp.float32)
    acc = step(0, 0, False, False, acc)
    acc = step(1, 1, False, True, acc)
    acc = step(2, 0, True, True, acc)

    def pair(j, acc):
        acc = step(2 * j + 3, 1, True, True, acc)
        acc = step(2 * j + 4, 0, True, True, acc)
        return acc

    acc = lax.fori_loop(0, (NBLK - 4) // 2, pair, acc)

    acc = step(NBLK - 1, 1, True, False, acc)
    acc_v[...] = acc
    pltpu.sync_copy(acc_v, partials.at[pl.ds(wid * L, L)])
    pltpu.make_async_copy(*s_desc(NBLK - 2, 0)).wait()
    pltpu.make_async_copy(*s_desc(NBLK - 1, 1)).wait()


def _sc_main(table3, flat_idx, flat_tgt, lse):
    mesh = plsc.VectorSubcoreMesh(core_axis_name="c", subcore_axis_name="s")
    f = functools.partial(
        pl.kernel,
        mesh=mesh,
        compiler_params=pltpu.CompilerParams(needs_layout_passes=False),
        out_type=[
            jax.ShapeDtypeStruct((TOKENS, VOCAB), jnp.float32),
            jax.ShapeDtypeStruct((NW * L,), jnp.float32),
        ],
        scratch_types=[
            pltpu.VMEM((TPW,), jnp.int32),
            pltpu.VMEM((TPW,), jnp.int32),
            pltpu.VMEM((VOCAB,), jnp.float32),
            pltpu.VMEM((NB, 8, 128), jnp.float32),
            pltpu.VMEM((NB, 8, 128), jnp.float32),
            pltpu.VMEM((NB, VOCAB), jnp.float32),
            pltpu.VMEM((NB, VOCAB), jnp.float32),
            pltpu.VMEM((L,), jnp.float32),
            pltpu.SemaphoreType.DMA,
            pltpu.SemaphoreType.DMA,
            pltpu.SemaphoreType.DMA,
            pltpu.SemaphoreType.DMA,
        ],
    )(_sc_body)
    return f(table3, flat_idx, flat_tgt, lse)


def kernel(idx, targets, token_embedding_table):
    flat_idx = idx.reshape(-1).astype(jnp.int32)
    flat_tgt = targets.reshape(-1).astype(jnp.int32)
    table3 = jnp.pad(
        token_embedding_table, ((0, 0), (0, VPAD - VOCAB))
    ).reshape(VOCAB, 8, 128)
    lse = _lse_table(token_embedding_table)
    flat_logits, partials = _sc_main(table3, flat_idx, flat_tgt, lse)
    loss = jnp.sum(partials) / TOKENS
    return (flat_logits, loss)


# speedup vs baseline: 1.0230x; 1.0008x over previous
import functools

import jax
import jax.numpy as jnp
from jax import lax
from jax.experimental import pallas as pl
from jax.experimental.pallas import tpu as pltpu
from jax.experimental.pallas import tpu_sc as plsc

VOCAB = 1000
VPAD = 1024
TOKENS = 1024 * 50

NC = 2
NS = 16
NW = NC * NS
L = 16
TPW = TOKENS // NW
NB = 16
NBLK = TPW // NB


def _lse_body(t_ref, o_ref, t3_ref):
    t = t_ref[...]
    m = jnp.max(t, axis=1, keepdims=True)
    s = jnp.sum(jnp.exp(t - m), axis=1)
    o_ref[...] = m[:, 0] + jnp.log(s)
    for sb in range(7):
        t3_ref[:, sb, :] = t[:, sb * 128:(sb + 1) * 128]
    t3_ref[:, 7, :] = jnp.concatenate(
        [t[:, 896:VOCAB], jnp.zeros((VOCAB, VPAD - VOCAB), jnp.float32)], 1)


def _lse_table(table):
    return pl.pallas_call(
        _lse_body,
        out_shape=[
            jax.ShapeDtypeStruct((VOCAB,), jnp.float32),
            jax.ShapeDtypeStruct((VOCAB, 8, 128), jnp.float32),
        ],
    )(table)


def _sc_body(table3, idxs, tgts, lse, out, partials,
             idx_v, tgt_v, lse_v,
             r3_0, r3_1, rk_0, rk_1, acc_v, g0, g1, o0, o1):
    wid = lax.axis_index("s") * NC + lax.axis_index("c")
    base = pl.multiple_of(wid * TPW, 8)
    pltpu.sync_copy(idxs.at[pl.ds(base, TPW)], idx_v)
    pltpu.sync_copy(tgts.at[pl.ds(base, TPW)], tgt_v)
    pltpu.sync_copy(lse, lse_v)

    rows3 = (r3_0, r3_1)
    rowsk = (rk_0, rk_1)
    gs = (g0, g1)
    os_ = (o0, o1)
    tok16 = lax.iota(jnp.int32, L)

    def g_desc(b, buf):
        src = table3.at[idx_v.at[pl.ds(pl.multiple_of(b * NB, 8), NB)]]
        return src, rows3[buf], gs[buf]

    def s_desc(b, buf):
        dst = out.at[pl.ds(pl.multiple_of(base + b * NB, 8), NB)]
        return rowsk[buf], dst, os_[buf]

    pltpu.async_copy(*g_desc(0, 0))
    pltpu.async_copy(*g_desc(1, 1))

    def rechunk(buf):
        r3 = rows3[buf]
        rk = rowsk[buf]

        for t in range(NB):
            for ch in range(62):
                rk[t, pl.ds(ch * L, L)] = r3[t, ch >> 3, pl.ds((ch & 7) * L, L)]
            rk[t, pl.ds(984, L)] = r3[t, 7, pl.ds(88, L)]

    def step(b, buf, wait_s2, start_next, acc):
        pltpu.make_async_copy(*g_desc(b, buf)).wait()
        if start_next:
            pltpu.async_copy(*g_desc(b + 1, 1 - buf))
        if wait_s2:
            pltpu.make_async_copy(*s_desc(b - 2, buf)).wait()
        off = pl.multiple_of(b * L, 8)
        i16 = idx_v[pl.ds(off, L)]
        t16 = tgt_v[pl.ds(off, L)]
        l16 = plsc.load_gather(lse_v, [i16])
        p16 = plsc.load_gather(
            rows3[buf],
            [tok16, lax.shift_right_logical(t16, 7), lax.bitwise_and(t16, 127)])
        acc = acc + (l16 - p16)
        rechunk(buf)
        pltpu.async_copy(*s_desc(b, buf))
        return acc

    acc = jnp.zeros((L,), jnp.float32)
    acc = step(0, 0, False, False, acc)
    acc = step(1, 1, False, True, acc)
    acc = step(2, 0, True, True, acc)

    def pair(j, acc):
        acc = step(2 * j + 3, 1, True, True, acc)
        acc = step(2 * j + 4, 0, True, True, acc)
        return acc

    acc = lax.fori_loop(0, (NBLK - 4) // 2, pair, acc)

    acc = step(NBLK - 1, 1, True, False, acc)
    acc_v[...] = acc
    pltpu.sync_copy(acc_v, partials.at[pl.ds(wid * L, L)])
    pltpu.make_async_copy(*s_desc(NBLK - 2, 0)).wait()
    pltpu.make_async_copy(*s_desc(NBLK - 1, 1)).wait()


def _sc_main(table3, flat_idx, flat_tgt, lse):
    mesh = plsc.VectorSubcoreMesh(core_axis_name="c", subcore_axis_name="s")
    f = functools.partial(
        pl.kernel,
        mesh=mesh,
        compiler_params=pltpu.CompilerParams(needs_layout_passes=False),
        out_type=[
            jax.ShapeDtypeStruct((TOKENS, VOCAB), jnp.float32),
            jax.ShapeDtypeStruct((NW * L,), jnp.float32),
        ],
        scratch_types=[
            pltpu.VMEM((TPW,), jnp.int32),
            pltpu.VMEM((TPW,), jnp.int32),
            pltpu.VMEM((VOCAB,), jnp.float32),
            pltpu.VMEM((NB, 8, 128), jnp.float32),
            pltpu.VMEM((NB, 8, 128), jnp.float32),
            pltpu.VMEM((NB, VOCAB), jnp.float32),
            pltpu.VMEM((NB, VOCAB), jnp.float32),
            pltpu.VMEM((L,), jnp.float32),
            pltpu.SemaphoreType.DMA,
            pltpu.SemaphoreType.DMA,
            pltpu.SemaphoreType.DMA,
            pltpu.SemaphoreType.DMA,
        ],
    )(_sc_body)
    return f(table3, flat_idx, flat_tgt, lse)


def kernel(idx, targets, token_embedding_table):
    flat_idx = idx.reshape(-1).astype(jnp.int32)
    flat_tgt = targets.reshape(-1).astype(jnp.int32)
    lse, table3 = _lse_table(token_embedding_table)
    flat_logits, partials = _sc_main(table3, flat_idx, flat_tgt, lse)
    loss = jnp.sum(partials) / TOKENS
    return (flat_logits, loss)
